# Initial kernel scaffold; baseline (speedup 1.0000x reference)
import jax
import jax.numpy as jnp
from jax.experimental import pallas as pl


def kernel(x, tables):
    raise NotImplementedError("write your pallas kernel here")


# speedup vs baseline: 1.0190x; 1.0190x over previous
import functools

import jax
import jax.numpy as jnp
from jax import lax
from jax.experimental import pallas as pl
from jax.experimental.pallas import tpu as pltpu
from jax.experimental.pallas import tpu_sc as plsc

_N_FIELDS = 26
_VOCAB = 100000
_DIM = 16
_BATCH = 16384
_LANES = 16

_NC = 2
_NS = 16
_NW = _NC * _NS
_BW = _BATCH // _NW
_CH = 64
_NCHUNK = _BW // _CH
_EPC = _CH * _N_FIELDS
_GROWS = 128
_GPC = _EPC // _GROWS
_VPC = _EPC // _LANES
_OFFLEN = 208


def _body(tab_hbm, x_hbm, off_hbm, out_hbm, x_v, off_v, idx_v, rows_v, out_v, sem):
    wid = lax.axis_index("s") * _NC + lax.axis_index("c")
    base = wid * _BW
    pltpu.sync_copy(x_hbm.at[pl.ds(base * _N_FIELDS, _BW * _N_FIELDS)], x_v)
    pltpu.sync_copy(off_hbm, off_v)

    def chunk(g, carry):
        def mk_idx(v, c):
            idx_v[v // 8, pl.ds((v % 8) * _LANES, _LANES)] = (
                x_v[pl.ds(g * _EPC + v * _LANES, _LANES)]
                + off_v[pl.ds((v % 13) * _LANES, _LANES)]
            )
            return c

        lax.fori_loop(0, _VPC, mk_idx, 0)

        copies = []
        for j in range(_GPC):
            copies.append(pltpu.async_copy(
                tab_hbm.at[idx_v.at[j]],
                rows_v.at[pl.ds(j * _GROWS, _GROWS)],
                sem,
            ))
        for cp in copies:
            cp.wait()

        def accum(i, c):
            r0 = i * _N_FIELDS
            acc = rows_v[r0, :]
            for f in range(1, _N_FIELDS):
                acc = acc + rows_v[r0 + f, :]
            out_v[g * _CH + i, :] = acc
            return c

        lax.fori_loop(0, _CH, accum, 0)
        return carry

    lax.fori_loop(0, _NCHUNK, chunk, 0)
    pltpu.sync_copy(out_v, out_hbm.at[pl.ds(base, _BW)])


@functools.lru_cache(maxsize=None)
def _build_emb():
    return functools.partial(
        pl.kernel,
        out_type=jax.ShapeDtypeStruct((_BATCH, _DIM), jnp.float32),
        mesh=plsc.VectorSubcoreMesh(core_axis_name="c", subcore_axis_name="s"),
        compiler_params=pltpu.CompilerParams(use_tc_tiling_on_sc=False),
        scratch_types=[
            pltpu.VMEM((_BW * _N_FIELDS,), jnp.int32),
            pltpu.VMEM((_OFFLEN,), jnp.int32),
            pltpu.VMEM((_GPC, _GROWS), jnp.int32),
            pltpu.VMEM((_EPC, _DIM), jnp.float32),
            pltpu.VMEM((_BW, _DIM), jnp.float32),
            pltpu.SemaphoreType.DMA,
        ],
    )(_body)


@jax.jit
def kernel(x, tables):
    tab = tables.reshape(_N_FIELDS * _VOCAB, _DIM)
    xf = x.reshape(-1)
    off = jnp.tile(jnp.arange(_N_FIELDS, dtype=jnp.int32) * _VOCAB,
                   _OFFLEN // _N_FIELDS)
    return _build_emb()(tab, xf, off)


# speedup vs baseline: 1.0293x; 1.0102x over previous
import functools

import jax
import jax.numpy as jnp
from jax import lax
from jax.experimental import pallas as pl
from jax.experimental.pallas import tpu as pltpu
from jax.experimental.pallas import tpu_sc as plsc

_N_FIELDS = 26
_VOCAB = 100000
_DIM = 16
_BATCH = 16384
_LANES = 16

_NC = 2
_NS = 16
_NW = _NC * _NS
_BW = _BATCH // _NW
_CH = 128
_NCHUNK = _BW // _CH


def _body(tab_hbm, xt_hbm, out_hbm, x_v, rows_v, out_v, sem):
    wid = lax.axis_index("s") * _NC + lax.axis_index("c")
    base = wid * _BW
    pltpu.sync_copy(xt_hbm.at[:, pl.ds(base, _BW)], x_v)

    def chunk(g, carry):
        copies = []
        for f in range(_N_FIELDS):
            copies.append(pltpu.async_copy(
                tab_hbm.at[f].at[x_v.at[f, pl.ds(g * _CH, _CH)]],
                rows_v.at[f],
                sem,
            ))
        for cp in copies:
            cp.wait()

        def accum(i, c):
            acc = rows_v[0, i, :]
            for f in range(1, _N_FIELDS):
                acc = acc + rows_v[f, i, :]
            out_v[g * _CH + i, :] = acc
            return c

        lax.fori_loop(0, _CH, accum, 0)
        return carry

    lax.fori_loop(0, _NCHUNK, chunk, 0)
    pltpu.sync_copy(out_v, out_hbm.at[pl.ds(base, _BW)])


@functools.lru_cache(maxsize=None)
def _build_emb():
    return functools.partial(
        pl.kernel,
        out_type=jax.ShapeDtypeStruct((_BATCH, _DIM), jnp.float32),
        mesh=plsc.VectorSubcoreMesh(core_axis_name="c", subcore_axis_name="s"),
        compiler_params=pltpu.CompilerParams(use_tc_tiling_on_sc=False),
        scratch_types=[
            pltpu.VMEM((_N_FIELDS, _BW), jnp.int32),
            pltpu.VMEM((_N_FIELDS, _CH, _DIM), jnp.float32),
            pltpu.VMEM((_BW, _DIM), jnp.float32),
            pltpu.SemaphoreType.DMA,
        ],
    )(_body)


@jax.jit
def kernel(x, tables):
    return _build_emb()(tables, x.T, )
